# baseline (speedup 1.0000x reference)
import functools

import jax
import jax.numpy as jnp
from jax import lax
from jax.experimental import pallas as pl
from jax.experimental.pallas import tpu as pltpu
from jax.experimental.pallas import tpu_sc as plsc

W = 5
D = 64
B = 1024
L = 200
N = B * L


V = 1000000


def _sc_pad(table):
    info = plsc.get_sparse_core_info()
    nw = info.num_cores * info.num_subcores
    chunk = 1600
    n_chunks = V // chunk

    mesh = plsc.VectorSubcoreMesh(core_axis_name="c", subcore_axis_name="s")

    @functools.partial(
        pl.kernel,
        out_type=jax.ShapeDtypeStruct((V, 2 * D), jnp.float32),
        mesh=mesh,
        scratch_types=[pltpu.VMEM((chunk, D), jnp.float32)],
    )
    def pad_kernel(table_hbm, out_hbm, buf_v):
        wid = lax.axis_index("s") * info.num_cores + lax.axis_index("c")

        def body(i, carry):
            c = wid + nw * i

            @pl.when(c < n_chunks)
            def _():
                base = c * chunk
                pltpu.sync_copy(table_hbm.at[pl.ds(base, chunk), :], buf_v)
                pltpu.sync_copy(
                    buf_v, out_hbm.at[pl.ds(base, chunk), pl.ds(0, D)]
                )

            return carry

        lax.fori_loop(0, -(-n_chunks // nw), body, 0)

    return pad_kernel(table)


_CH = 256


def _sc_gather_t(idx_flat, table_pad):
    info = plsc.get_sparse_core_info()
    nw = info.num_cores * info.num_subcores
    n_chunks = 4 * L
    n_iter = -(-n_chunks // nw)

    mesh = plsc.VectorSubcoreMesh(core_axis_name="c", subcore_axis_name="s")

    @functools.partial(
        pl.kernel,
        out_type=jax.ShapeDtypeStruct(((L + W - 1) * D, B), jnp.float32),
        mesh=mesh,
        scratch_types=[
            pltpu.VMEM((_CH,), jnp.int32),
            pltpu.VMEM((_CH, 2 * D), jnp.float32),
            pltpu.VMEM((_CH, 2 * D + 1), jnp.float32),
            pltpu.VMEM((D, _CH), jnp.float32),
            pltpu.SemaphoreType.DMA,
        ],
        compiler_params=pltpu.CompilerParams(needs_layout_passes=False),
    )
    def gather_kernel(
        table_hbm, idx_hbm, out_hbm, idx_v, rows_v, wide_v, slab_v, sem
    ):
        wid = lax.axis_index("s") * info.num_cores + lax.axis_index("c")

        @pl.when(wid < 4 * (W - 1))
        def _():
            def zfill_d(d, carry):
                for g in range(_CH // 16):
                    slab_v[d, pl.ds(g * 16, 16)] = jnp.zeros((16,), jnp.float32)
                return carry

            lax.fori_loop(0, D, zfill_d, 0)
            tz = L + wid // 4
            bz = (wid % 4) * _CH
            pltpu.sync_copy(slab_v, out_hbm.at[pl.ds(tz * D, D), pl.ds(bz, _CH)])

        def body(i, carry):
            c = wid + nw * i

            @pl.when(c < n_chunks)
            def _():
                j = c // 4
                b0 = (c % 4) * _CH
                t = (L - 1) - j
                pltpu.sync_copy(idx_hbm.at[pl.ds(c * _CH, _CH)], idx_v)
                pltpu.async_copy(table_hbm.at[idx_v], rows_v, sem).wait()

                def stage_r(r, carry2):
                    for g in range(D // 16):
                        wide_v[r, pl.ds(g * 16, 16)] = rows_v[r, pl.ds(g * 16, 16)]
                    return carry2

                lax.fori_loop(0, _CH, stage_r, 0)

                lanes = lax.iota(jnp.int32, 16)

                def trans_d(d, carry2):
                    dcol = jnp.full((16,), d, jnp.int32)
                    for g in range(_CH // 16):
                        vals = plsc.load_gather(wide_v, [lanes + g * 16, dcol])
                        slab_v[d, pl.ds(g * 16, 16)] = vals
                    return carry2

                lax.fori_loop(0, D, trans_d, 0)
                pltpu.sync_copy(
                    slab_v, out_hbm.at[pl.ds(t * D, D), pl.ds(b0, _CH)]
                )

            return carry

        lax.fori_loop(0, n_iter, body, 0)

    return gather_kernel(table_pad, idx_flat)


def _sc_window(embt2):
    info = plsc.get_sparse_core_info()
    nw = info.num_cores * info.num_subcores
    n_slabs = L + W - 1
    per_w = -(-n_slabs // nw)

    mesh = plsc.VectorSubcoreMesh(core_axis_name="c", subcore_axis_name="s")

    @functools.partial(
        pl.kernel,
        out_type=jax.ShapeDtypeStruct((L, W * D, B), jnp.float32),
        mesh=mesh,
        scratch_types=[pltpu.VMEM((D, B), jnp.float32)],
    )
    def window_kernel(embt_hbm, out_hbm, slab_v):
        wid = lax.axis_index("s") * info.num_cores + lax.axis_index("c")

        def body(i, carry):
            t = wid + nw * i

            @pl.when(t < n_slabs)
            def _():
                pltpu.sync_copy(embt_hbm.at[pl.ds(t * D, D)], slab_v)
                for k in range(W):
                    j = L - 1 - t + k

                    @pl.when((k <= t) & (t - (L - 1) <= k))
                    def _():
                        pltpu.sync_copy(
                            slab_v, out_hbm.at[j, pl.ds(k * D, D)]
                        )

            return carry

        lax.fori_loop(0, per_w, body, 0)

    return window_kernel(embt2)


def kernel(inputs, table):
    table_pad = jnp.pad(table, ((0, 0), (0, 2 * D - D)))
    idxt_flat = inputs.T.reshape(-1).astype(jnp.int32)
    embt2 = _sc_gather_t(idxt_flat, table_pad)
    p = _sc_window(embt2)
    return p.transpose(2, 0, 1)


# speedup vs baseline: 1.2889x; 1.2889x over previous
import functools

import jax
import jax.numpy as jnp
from jax import lax
from jax.experimental import pallas as pl
from jax.experimental.pallas import tpu as pltpu
from jax.experimental.pallas import tpu_sc as plsc

W = 5
D = 64
B = 1024
L = 200
N = B * L


def _sc_gather(idx_flat, table_pad):
    info = plsc.get_sparse_core_info()
    nw = info.num_cores * info.num_subcores
    per_w = N // nw
    chunk = 800
    n_chunks = per_w // chunk

    mesh = plsc.VectorSubcoreMesh(core_axis_name="c", subcore_axis_name="s")

    @functools.partial(
        pl.kernel,
        out_type=jax.ShapeDtypeStruct((N, 2 * D), jnp.float32),
        mesh=mesh,
        scratch_types=[
            pltpu.VMEM((chunk,), jnp.int32),
            pltpu.VMEM((chunk, 2 * D), jnp.float32),
            pltpu.SemaphoreType.DMA,
        ],
    )
    def gather_kernel(table_hbm, idx_hbm, out_hbm, idx_v, rows_v, sem):
        wid = lax.axis_index("s") * info.num_cores + lax.axis_index("c")

        def body(i, carry):
            base = wid * per_w + i * chunk
            pltpu.sync_copy(idx_hbm.at[pl.ds(base, chunk)], idx_v)
            pltpu.async_copy(table_hbm.at[idx_v], rows_v, sem).wait()
            pltpu.sync_copy(rows_v, out_hbm.at[pl.ds(base, chunk)])
            return carry

        lax.fori_loop(0, n_chunks, body, 0)

    return gather_kernel(table_pad, idx_flat)


def _transpose_body(embj_ref, embt_ref):
    t = pl.program_id(0)

    @pl.when(t <= L - 1)
    def _():
        x = embj_ref[0, :, :D]
        embt_ref[...] = jnp.transpose(x, (1, 0))

    @pl.when(t > L - 1)
    def _():
        embt_ref[...] = jnp.zeros((D, B), jnp.float32)


def _tc_transpose(embj3):
    return pl.pallas_call(
        _transpose_body,
        grid=(L + W - 1,),
        in_specs=[
            pl.BlockSpec((1, B, 2 * D), lambda t: (jnp.maximum(L - 1 - t, 0), 0, 0))
        ],
        out_specs=pl.BlockSpec((D, B), lambda t: (t, 0)),
        out_shape=jax.ShapeDtypeStruct(((L + W - 1) * D, B), jnp.float32),
    )(embj3)


def _sc_window(embt2):
    info = plsc.get_sparse_core_info()
    nw = info.num_cores * info.num_subcores
    n_slabs = L + W - 1
    per_w = -(-n_slabs // nw)

    mesh = plsc.VectorSubcoreMesh(core_axis_name="c", subcore_axis_name="s")

    @functools.partial(
        pl.kernel,
        out_type=jax.ShapeDtypeStruct((L, W * D, B), jnp.float32),
        mesh=mesh,
        scratch_types=[pltpu.VMEM((D, B), jnp.float32)],
    )
    def window_kernel(embt_hbm, out_hbm, slab_v):
        wid = lax.axis_index("s") * info.num_cores + lax.axis_index("c")

        def body(i, carry):
            t = wid + nw * i

            @pl.when(t < n_slabs)
            def _():
                pltpu.sync_copy(embt_hbm.at[pl.ds(t * D, D)], slab_v)
                for k in range(W):
                    j = L - 1 - t + k

                    @pl.when((k <= t) & (t - (L - 1) <= k))
                    def _():
                        pltpu.sync_copy(
                            slab_v, out_hbm.at[j, pl.ds(k * D, D)]
                        )

            return carry

        lax.fori_loop(0, per_w, body, 0)

    return window_kernel(embt2)


def kernel(inputs, table):
    table_pad = jnp.pad(table, ((0, 0), (0, 2 * D - D)))
    idxt_flat = inputs.T.reshape(-1).astype(jnp.int32)
    embj = _sc_gather(idxt_flat, table_pad)
    embj3 = embj.reshape(L, B, 2 * D)
    embt2 = _tc_transpose(embj3)
    p = _sc_window(embt2)
    return p.transpose(2, 0, 1)


# speedup vs baseline: 6.4213x; 4.9822x over previous
import functools

import jax
import jax.numpy as jnp
from jax import lax
from jax.experimental import pallas as pl
from jax.experimental.pallas import tpu as pltpu
from jax.experimental.pallas import tpu_sc as plsc

W = 5
D = 64
B = 1024
L = 200
N = B * L


def _sc_gather(idx_flat, table_pad):
    info = plsc.get_sparse_core_info()
    nw = info.num_cores * info.num_subcores
    per_w = N // nw
    chunk = 800
    n_chunks = per_w // chunk

    mesh = plsc.VectorSubcoreMesh(core_axis_name="c", subcore_axis_name="s")

    @functools.partial(
        pl.kernel,
        out_type=jax.ShapeDtypeStruct((N, 2 * D), jnp.float32),
        mesh=mesh,
        scratch_types=[
            pltpu.VMEM((chunk,), jnp.int32),
            pltpu.VMEM((chunk, 2 * D), jnp.float32),
            pltpu.SemaphoreType.DMA,
        ],
    )
    def gather_kernel(table_hbm, idx_hbm, out_hbm, idx_v, rows_v, sem):
        wid = lax.axis_index("s") * info.num_cores + lax.axis_index("c")

        def body(i, carry):
            base = wid * per_w + i * chunk
            pltpu.sync_copy(idx_hbm.at[pl.ds(base, chunk)], idx_v)
            pltpu.async_copy(table_hbm.at[idx_v], rows_v, sem).wait()
            pltpu.sync_copy(rows_v, out_hbm.at[pl.ds(base, chunk)])
            return carry

        lax.fori_loop(0, n_chunks, body, 0)

    return gather_kernel(table_pad, idx_flat)


def _transpose_body(embj_ref, embt_ref):
    t = pl.program_id(0)

    @pl.when(t <= L - 1)
    def _():
        x = embj_ref[0, :, :D]
        embt_ref[...] = jnp.transpose(x, (1, 0))

    @pl.when(t > L - 1)
    def _():
        embt_ref[...] = jnp.zeros((D, B), jnp.float32)


def _tc_transpose(embj3):
    return pl.pallas_call(
        _transpose_body,
        grid=(L + W - 1,),
        in_specs=[
            pl.BlockSpec((1, B, 2 * D), lambda t: (jnp.maximum(L - 1 - t, 0), 0, 0))
        ],
        out_specs=pl.BlockSpec((D, B), lambda t: (t, 0)),
        out_shape=jax.ShapeDtypeStruct(((L + W - 1) * D, B), jnp.float32),
    )(embj3)


def _sc_window(embt2):
    info = plsc.get_sparse_core_info()
    nw = info.num_cores * info.num_subcores
    n_slabs = L + W - 1
    per_w = -(-n_slabs // nw)

    mesh = plsc.VectorSubcoreMesh(core_axis_name="c", subcore_axis_name="s")

    @functools.partial(
        pl.kernel,
        out_type=jax.ShapeDtypeStruct((L, W * D, B), jnp.float32),
        mesh=mesh,
        scratch_types=[
            pltpu.VMEM((D, B), jnp.float32),
            pltpu.SemaphoreType.DMA,
        ],
    )
    def window_kernel(embt_hbm, out_hbm, slab_v, wsem):
        wid = lax.axis_index("s") * info.num_cores + lax.axis_index("c")

        def body(i, carry):
            t = wid + nw * i

            @pl.when(t < n_slabs)
            def _():
                pltpu.sync_copy(embt_hbm.at[pl.ds(t * D, D)], slab_v)
                for k in range(W):
                    j = L - 1 - t + k

                    @pl.when((k <= t) & (t - (L - 1) <= k))
                    def _():
                        pltpu.async_copy(
                            slab_v, out_hbm.at[j, pl.ds(k * D, D)], wsem
                        ).start()

                for k in range(W):
                    j = L - 1 - t + k

                    @pl.when((k <= t) & (t - (L - 1) <= k))
                    def _():
                        pltpu.async_copy(
                            slab_v, out_hbm.at[j, pl.ds(k * D, D)], wsem
                        ).wait()

            return carry

        lax.fori_loop(0, per_w, body, 0)

    return window_kernel(embt2)


def kernel(inputs, table):
    table_pad = jnp.pad(table, ((0, 0), (0, 2 * D - D)))
    idxt_flat = inputs.T.reshape(-1).astype(jnp.int32)
    embj = _sc_gather(idxt_flat, table_pad)
    embj3 = embj.reshape(L, B, 2 * D)
    embt2 = _tc_transpose(embj3)
    p = _sc_window(embt2)
    return p.transpose(2, 0, 1)
